# baseline (speedup 1.0000x reference)
import functools
import math

import jax
import jax.numpy as jnp
import numpy as np
from jax import lax
from jax.experimental import pallas as pl
from jax.experimental.pallas import tpu as pltpu
from jax.experimental.pallas import tpu_sc as plsc

H = 128
G = 4
GF = 32
GC = 3 * GF
HW = GC // 2
MROW = 4 * GF
NC, NS = 2, 16
CHUNK = 80


def _pack_words(a_bits, b_bits):
    lo = lax.shift_right_logical(a_bits + jnp.int32(0x8000), 16)
    hi = (b_bits + jnp.int32(0x8000)) & jnp.int32(-65536)
    return hi | lo


def _node_mlp_body(x_ref, w1_ref, b1_ref, w2_ref, b2_ref, o_ref):
    xb = x_ref[...]
    h = jnp.dot(xb, w1_ref[...].T, preferred_element_type=jnp.float32)
    h = h + b1_ref[...]
    h = (h * jax.nn.sigmoid(h)) * (1.0 / 0.6)
    y = jnp.dot(h, w2_ref[...].T, preferred_element_type=jnp.float32)
    o_ref[...] = y + b2_ref[...]


def _edge_proj_body(r_ref, ev_ref, we_ref, be_ref, o_ref):
    r = r_ref[...].astype(jnp.bfloat16)
    evb = lax.bitcast_convert_type(ev_ref[...], jnp.int32)
    for g in range(G):
        ya = jnp.dot(r, we_ref[g, 0].T, preferred_element_type=jnp.float32)
        yb = jnp.dot(r, we_ref[g, 1].T, preferred_element_type=jnp.float32)
        ya = lax.bitcast_convert_type(ya + be_ref[g, 0], jnp.int32)
        yb = lax.bitcast_convert_type(yb + be_ref[g, 1], jnp.int32)
        o_ref[g, :, 0:HW] = _pack_words(ya, yb)
        o_ref[g, :, HW:HW + 16] = evb


def _node_mlp(x, W1, b1, W2P, b2P):
    n = x.shape[0]
    bn = 2000
    return pl.pallas_call(
        _node_mlp_body,
        grid=(n // bn,),
        in_specs=[
            pl.BlockSpec((bn, H), lambda m: (m, 0)),
            pl.BlockSpec(W1.shape, lambda m: (0, 0)),
            pl.BlockSpec((1, H // 2), lambda m: (0, 0)),
            pl.BlockSpec(W2P.shape, lambda m: (0, 0)),
            pl.BlockSpec((1, 3 * H), lambda m: (0, 0)),
        ],
        out_specs=pl.BlockSpec((bn, 3 * H), lambda m: (m, 0)),
        out_shape=jax.ShapeDtypeStruct((n, 3 * H), jnp.float32),
    )(x, W1, b1.reshape(1, -1), W2P, b2P.reshape(1, -1))


def _edge_proj(edge_rbf, ev16, WeP, beP):
    e = edge_rbf.shape[0]
    be_blk = 4000
    weab = WeP.astype(jnp.bfloat16).reshape(G, 2, HW, WeP.shape[1])
    beab = beP.reshape(G, 2, 1, HW)
    return pl.pallas_call(
        _edge_proj_body,
        grid=(e // be_blk,),
        in_specs=[
            pl.BlockSpec((be_blk, edge_rbf.shape[1]), lambda m: (m, 0)),
            pl.BlockSpec((be_blk, 16), lambda m: (m, 0)),
            pl.BlockSpec(weab.shape, lambda m: (0, 0, 0, 0)),
            pl.BlockSpec(beab.shape, lambda m: (0, 0, 0, 0)),
        ],
        out_specs=pl.BlockSpec((G, be_blk, H), lambda m: (0, m, 0)),
        out_shape=jax.ShapeDtypeStruct((G, e, H), jnp.int32),
    )(edge_rbf, ev16, weab, beab)


def _sc_body(n_pad, n_edges,
             t2_hbm, rbf_hbm, ej_hbm, ei_hbm, out_hbm,
             acc,
             idxj0, idxj1, idxt0, idxt1, idxi0, idxi1,
             tv0, tv1, rbfv0, rbfv1, msg_v, zrow_v,
             semg0, semg1, semr0, semr1, semi0, semi1):
    c = lax.axis_index("c")
    s = lax.axis_index("s")
    idxj = (idxj0, idxj1)
    idxt = (idxt0, idxt1)
    idxi = (idxi0, idxi1)
    tv = (tv0, tv1)
    rbfv = (rbfv0, rbfv1)
    semg = (semg0, semg1)
    semr = (semr0, semr1)
    semi = (semi0, semi1)

    e_per_tile = n_edges // NS
    n_chunks = e_per_tile // CHUNK
    rows_per_tile = n_pad // NS
    zrows = zrow_v.shape[0]
    n_zcopies = rows_per_tile // zrows
    ebase = s * e_per_tile
    row0 = s * rows_per_tile

    def _e0(ci):
        return pl.multiple_of(ebase + ci * CHUNK, 8)

    def issue_idx(ci, b):
        pltpu.async_copy(ej_hbm.at[pl.ds(_e0(ci), CHUNK)], idxj[b], semi[b])
        pltpu.async_copy(ei_hbm.at[pl.ds(_e0(ci), CHUNK)], idxi[b], semi[b])

    def wait_idx(b):
        pltpu.make_async_copy(ej_hbm.at[pl.ds(0, CHUNK)], idxj[b],
                              semi[b]).wait()
        pltpu.make_async_copy(ei_hbm.at[pl.ds(0, CHUNK)], idxi[b],
                              semi[b]).wait()

    def issue_main(ci, b, g):
        def _mkidx(k, _):
            idxt[b][pl.ds(k * 16, 16)] = idxj[b][pl.ds(k * 16, 16)] * G + g
            return 0
        lax.fori_loop(0, CHUNK // 16, _mkidx, 0)
        pltpu.async_copy(t2_hbm.at[idxt[b]], tv[b], semg[b])
        pltpu.async_copy(rbf_hbm.at[g, pl.ds(_e0(ci), CHUNK), pl.ds(0, 64)],
                         rbfv[b], semr[b])

    def wait_main(b):
        pltpu.make_async_copy(t2_hbm.at[idxt[b]], tv[b], semg[b]).wait()
        pltpu.make_async_copy(rbf_hbm.at[0, pl.ds(0, CHUNK), pl.ds(0, 64)],
                              rbfv[b], semr[b]).wait()

    def compute_scatter(b):
        def _edge_pair(e2, _):
            for _o in range(2):
                _edge_one(e2 * 2 + _o)
            return 0

        def _edge_one(e):
            def _split(w):
                lo = lax.bitcast_convert_type(w << 16, jnp.float32)
                hi = lax.bitcast_convert_type(w & jnp.int32(-65536),
                                              jnp.float32)
                return lo, hi

            evw = lax.bitcast_convert_type(rbfv[b][e, pl.ds(48, 16)],
                                           jnp.float32)
            ev0 = evw[0]
            ev1 = evw[1]
            ev2 = evw[2]
            xh1l, xh1h = _split(tv[b][e, pl.ds(0, 16)])
            xh2l, xh2h = _split(tv[b][e, pl.ds(16, 16)])
            xh3l, xh3h = _split(tv[b][e, pl.ds(32, 16)])
            v0l, v0h = _split(tv[b][e, pl.ds(48, 16)])
            v1l, v1h = _split(tv[b][e, pl.ds(64, 16)])
            v2l, v2h = _split(tv[b][e, pl.ds(80, 16)])
            rb1l, rb1h = _split(rbfv[b][e, pl.ds(0, 16)])
            rb2l, rb2h = _split(rbfv[b][e, pl.ds(16, 16)])
            rb3l, rb3h = _split(rbfv[b][e, pl.ds(32, 16)])
            t1l = xh1l * rb1l
            t1h = xh1h * rb1h
            t2l = xh2l * rb2l
            t2h = xh2h * rb2h
            msg_v[e, pl.ds(0, 16)] = xh3l * rb3l
            msg_v[e, pl.ds(16, 16)] = xh3h * rb3h
            msg_v[e, pl.ds(32, 16)] = t1l * v0l + t2l * ev0
            msg_v[e, pl.ds(48, 16)] = t1h * v0h + t2h * ev0
            msg_v[e, pl.ds(64, 16)] = t1l * v1l + t2l * ev1
            msg_v[e, pl.ds(80, 16)] = t1h * v1h + t2h * ev1
            msg_v[e, pl.ds(96, 16)] = t1l * v2l + t2l * ev2
            msg_v[e, pl.ds(112, 16)] = t1h * v2h + t2h * ev2

        lax.fori_loop(0, CHUNK // 2, _edge_pair, 0)
        pltpu.sync_copy(msg_v, acc.at[idxi[b]], add=True)

    def _zfill16(k, _):
        r = k // (MROW // 16)
        col = (k % (MROW // 16)) * 16
        zrow_v[r, pl.ds(col, 16)] = jnp.zeros((16,), jnp.float32)
        return 0
    lax.fori_loop(0, zrows * (MROW // 16), _zfill16, 0)

    for p in range(2):
        g = c * 2 + p

        def _zero(k, _):
            pltpu.sync_copy(zrow_v, acc.at[pl.ds(row0 + k * zrows, zrows), :])
            return 0
        lax.fori_loop(0, n_zcopies, _zero, 0)
        plsc.subcore_barrier()

        issue_idx(0, 0)
        wait_idx(0)
        issue_main(0, 0, g)
        issue_idx(1, 1)

        def _pair(pi, _):
            for bb in range(2):
                ci = pi * 2 + bb
                wait_idx(1 - bb)
                issue_main(ci + 1, 1 - bb, g)
                wait_main(bb)
                compute_scatter(bb)

                @pl.when(ci < n_chunks - 2)
                def _():
                    issue_idx(ci + 2, bb)
            return 0
        lax.fori_loop(0, (n_chunks - 1) // 2, _pair, 0)

        last = (n_chunks - 1) % 2
        if n_chunks % 2 == 0:
            wait_idx(last)
            issue_main(n_chunks - 1, last, g)
            wait_main(1 - last)
            compute_scatter(1 - last)
        wait_main(last)
        compute_scatter(last)

        plsc.subcore_barrier()
        pltpu.sync_copy(acc.at[pl.ds(row0, rows_per_tile), :],
                        out_hbm.at[g, pl.ds(row0, rows_per_tile), :])
        plsc.subcore_barrier()


def _sc_call(t2, rbf_hp, ej, ei, n_pad, n_edges):
    mesh = plsc.VectorSubcoreMesh(core_axis_name="c", subcore_axis_name="s")
    body = functools.partial(_sc_body, n_pad, n_edges)
    dma = pltpu.SemaphoreType.DMA
    return pl.kernel(
        body,
        out_type=jax.ShapeDtypeStruct((G, n_pad, MROW), jnp.float32),
        mesh=mesh,
        compiler_params=pltpu.CompilerParams(use_tc_tiling_on_sc=False),
        scratch_types=[
            pltpu.VMEM_SHARED((n_pad, MROW), jnp.float32),
            pltpu.VMEM((CHUNK,), jnp.int32),
            pltpu.VMEM((CHUNK,), jnp.int32),
            pltpu.VMEM((CHUNK,), jnp.int32),
            pltpu.VMEM((CHUNK,), jnp.int32),
            pltpu.VMEM((CHUNK,), jnp.int32),
            pltpu.VMEM((CHUNK,), jnp.int32),
            pltpu.VMEM((CHUNK, H), jnp.int32),
            pltpu.VMEM((CHUNK, H), jnp.int32),
            pltpu.VMEM((CHUNK, 64), jnp.int32),
            pltpu.VMEM((CHUNK, 64), jnp.int32),
            pltpu.VMEM((CHUNK, MROW), jnp.float32),
            pltpu.VMEM((16, MROW), jnp.float32),
            dma, dma, dma, dma, dma, dma,
        ],
    )(t2, rbf_hp, ej, ei)


def _perm_and_scales():
    p = np.zeros(3 * H, dtype=np.int32)
    s = np.zeros(3 * H, dtype=np.float32)
    inv3 = 1.0 / math.sqrt(3.0)
    invh = 1.0 / math.sqrt(H)
    for g in range(G):
        for j in range(GC):
            half = j // HW
            c = (j % HW) // 16
            l = j % 16
            p[GC * g + j] = c * H + GF * g + 16 * half + l
            s[GC * g + j] = inv3 * invh if c < 2 else inv3
    return p, s


_P, _SFULL = _perm_and_scales()


def kernel(x, vec, edge_index, edge_rbf, edge_vector, W1, b1, W2, b2, We, be):
    n = x.shape[0]
    e = edge_rbf.shape[0]
    p = jnp.asarray(_P)
    sf = jnp.asarray(_SFULL)

    W2P, b2P = W2[p], b2[p]
    WeP = We[p] * sf[:, None]
    beP = be[p] * sf

    x_hp = _node_mlp(x, W1, b1, W2P, b2P)
    ev16 = jnp.pad(edge_vector, ((0, 0), (0, 13)))
    rbf_hp = _edge_proj(edge_rbf, ev16, WeP, beP)

    xb = lax.bitcast_convert_type(x_hp.reshape(n, G, 2, HW), jnp.int32)
    xw = _pack_words(xb[:, :, 0], xb[:, :, 1])
    vb = lax.bitcast_convert_type(vec, jnp.int32)
    vb = vb.reshape(n, 3, G, 2, 16)
    va = vb[:, :, :, 0].transpose(0, 2, 1, 3).reshape(n, G, HW)
    vbb = vb[:, :, :, 1].transpose(0, 2, 1, 3).reshape(n, G, HW)
    vw = _pack_words(va, vbb)
    pad = jnp.zeros((n, G, H - 2 * HW), jnp.int32)
    t2 = jnp.concatenate([xw, vw, pad], axis=2).reshape(n * G, H)

    ej = edge_index[0].astype(jnp.int32)
    ei = edge_index[1].astype(jnp.int32)

    n_pad = ((n + NS * 128 - 1) // (NS * 128)) * (NS * 128)
    outg = _sc_call(t2, rbf_hp, ej, ei, n_pad, e)
    outg = outg[:, :n].reshape(G, n, 4, GF)

    d_x = outg[:, :, 0, :].transpose(1, 0, 2).reshape(n, H)
    d_vec = (outg[:, :, 1:4, :].transpose(1, 2, 0, 3).reshape(n, 3, H))
    return (d_x, d_vec)


# speedup vs baseline: 1.0519x; 1.0519x over previous
import functools
import math

import jax
import jax.numpy as jnp
import numpy as np
from jax import lax
from jax.experimental import pallas as pl
from jax.experimental.pallas import tpu as pltpu
from jax.experimental.pallas import tpu_sc as plsc

H = 128
G = 4
GF = 32
GC = 3 * GF
HW = GC // 2
MROW = 4 * GF
NC, NS = 2, 16
CHUNK = 80


def _pack_words(a_bits, b_bits):
    lo = lax.shift_right_logical(a_bits + jnp.int32(0x8000), 16)
    hi = (b_bits + jnp.int32(0x8000)) & jnp.int32(-65536)
    return hi | lo


def _node_mlp_body(x_ref, w1_ref, b1_ref, w2_ref, b2_ref, o_ref):
    xb = x_ref[...]
    h = jnp.dot(xb, w1_ref[...].T, preferred_element_type=jnp.float32)
    h = h + b1_ref[...]
    h = (h * jax.nn.sigmoid(h)) * (1.0 / 0.6)
    y = jnp.dot(h, w2_ref[...].T, preferred_element_type=jnp.float32)
    o_ref[...] = y + b2_ref[...]


def _edge_proj_body(rl_ref, rh_ref, evl_ref, evh_ref, we_ref, be_ref, o_ref):
    bn = rl_ref.shape[0]
    r = jnp.concatenate([rl_ref[...], rh_ref[...]],
                        axis=0).astype(jnp.bfloat16)
    evl = lax.bitcast_convert_type(evl_ref[...], jnp.int32)
    evh = lax.bitcast_convert_type(evh_ref[...], jnp.int32)
    for g in range(G):
        ya = jnp.dot(r, we_ref[g, 0].T, preferred_element_type=jnp.float32)
        yb = jnp.dot(r, we_ref[g, 1].T, preferred_element_type=jnp.float32)
        ya = lax.bitcast_convert_type(ya + be_ref[g, 0], jnp.int32)
        yb = lax.bitcast_convert_type(yb + be_ref[g, 1], jnp.int32)
        w = _pack_words(ya, yb)
        o_ref[g, :, 0:HW] = w[0:bn]
        o_ref[g, :, HW:HW + 16] = evl
        o_ref[g, :, 64:64 + HW] = w[bn:2 * bn]
        o_ref[g, :, 64 + HW:128] = evh


def _node_mlp(x, W1, b1, W2P, b2P):
    n = x.shape[0]
    bn = 2000
    return pl.pallas_call(
        _node_mlp_body,
        grid=(n // bn,),
        in_specs=[
            pl.BlockSpec((bn, H), lambda m: (m, 0)),
            pl.BlockSpec(W1.shape, lambda m: (0, 0)),
            pl.BlockSpec((1, H // 2), lambda m: (0, 0)),
            pl.BlockSpec(W2P.shape, lambda m: (0, 0)),
            pl.BlockSpec((1, 3 * H), lambda m: (0, 0)),
        ],
        out_specs=pl.BlockSpec((bn, 3 * H), lambda m: (m, 0)),
        out_shape=jax.ShapeDtypeStruct((n, 3 * H), jnp.float32),
    )(x, W1, b1.reshape(1, -1), W2P, b2P.reshape(1, -1))


def _edge_proj(edge_rbf, ev16, WeP, beP):
    e = edge_rbf.shape[0]
    be_blk = 4000
    nb = e // 2 // be_blk
    weab = WeP.astype(jnp.bfloat16).reshape(G, 2, HW, WeP.shape[1])
    beab = beP.reshape(G, 2, 1, HW)
    return pl.pallas_call(
        _edge_proj_body,
        grid=(nb,),
        in_specs=[
            pl.BlockSpec((be_blk, edge_rbf.shape[1]), lambda m: (m, 0)),
            pl.BlockSpec((be_blk, edge_rbf.shape[1]), lambda m: (m + nb, 0)),
            pl.BlockSpec((be_blk, 16), lambda m: (m, 0)),
            pl.BlockSpec((be_blk, 16), lambda m: (m + nb, 0)),
            pl.BlockSpec(weab.shape, lambda m: (0, 0, 0, 0)),
            pl.BlockSpec(beab.shape, lambda m: (0, 0, 0, 0)),
        ],
        out_specs=pl.BlockSpec((G, be_blk, H), lambda m: (0, m, 0)),
        out_shape=jax.ShapeDtypeStruct((G, e // 2, H), jnp.int32),
    )(edge_rbf, edge_rbf, ev16, ev16, weab, beab)


def _sc_body(n_pad, n_edges,
             t2_hbm, rbf_hbm, ej_hbm, ei_hbm, out_hbm,
             acc,
             idxj0, idxj1, idxt0, idxt1, idxi0, idxi1,
             tv0, tv1, rbfv0, rbfv1, msg_v, zrow_v,
             semg0, semg1, semr0, semr1, semi0, semi1):
    c = lax.axis_index("c")
    s = lax.axis_index("s")
    idxj = (idxj0, idxj1)
    idxt = (idxt0, idxt1)
    idxi = (idxi0, idxi1)
    tv = (tv0, tv1)
    rbfv = (rbfv0, rbfv1)
    semg = (semg0, semg1)
    semr = (semr0, semr1)
    semi = (semi0, semi1)

    e_per_tile = n_edges // NS
    n_chunks = e_per_tile // CHUNK
    rows_per_tile = n_pad // NS
    zrows = zrow_v.shape[0]
    n_zcopies = rows_per_tile // zrows
    ebase = s * e_per_tile
    row0 = s * rows_per_tile

    def _e0(ci):
        return pl.multiple_of(ebase + ci * CHUNK, 8)

    rbf_rbase = pl.multiple_of((s % 8) * e_per_tile, 8)
    rbf_col = pl.multiple_of((s // 8) * 64, 64)

    def _r0(ci):
        return pl.multiple_of(rbf_rbase + ci * CHUNK, 8)

    def issue_idx(ci, b):
        pltpu.async_copy(ej_hbm.at[pl.ds(_e0(ci), CHUNK)], idxj[b], semi[b])
        pltpu.async_copy(ei_hbm.at[pl.ds(_e0(ci), CHUNK)], idxi[b], semi[b])

    def wait_idx(b):
        pltpu.make_async_copy(ej_hbm.at[pl.ds(0, CHUNK)], idxj[b],
                              semi[b]).wait()
        pltpu.make_async_copy(ei_hbm.at[pl.ds(0, CHUNK)], idxi[b],
                              semi[b]).wait()

    def issue_main(ci, b, g):
        def _mkidx(k, _):
            idxt[b][pl.ds(k * 16, 16)] = idxj[b][pl.ds(k * 16, 16)] * G + g
            return 0
        lax.fori_loop(0, CHUNK // 16, _mkidx, 0)
        pltpu.async_copy(t2_hbm.at[idxt[b]], tv[b], semg[b])
        pltpu.async_copy(
            rbf_hbm.at[g, pl.ds(_r0(ci), CHUNK), pl.ds(rbf_col, 64)],
            rbfv[b], semr[b])

    def wait_main(b):
        pltpu.make_async_copy(t2_hbm.at[idxt[b]], tv[b], semg[b]).wait()
        pltpu.make_async_copy(rbf_hbm.at[0, pl.ds(0, CHUNK), pl.ds(0, 64)],
                              rbfv[b], semr[b]).wait()

    def compute_scatter(b):
        def _edge(e, _):
            def _split(w):
                lo = lax.bitcast_convert_type(w << 16, jnp.float32)
                hi = lax.bitcast_convert_type(w & jnp.int32(-65536),
                                              jnp.float32)
                return lo, hi

            evw = lax.bitcast_convert_type(rbfv[b][e, pl.ds(48, 16)],
                                           jnp.float32)
            ev0 = evw[0]
            ev1 = evw[1]
            ev2 = evw[2]
            xh1l, xh1h = _split(tv[b][e, pl.ds(0, 16)])
            xh2l, xh2h = _split(tv[b][e, pl.ds(16, 16)])
            xh3l, xh3h = _split(tv[b][e, pl.ds(32, 16)])
            v0l, v0h = _split(tv[b][e, pl.ds(48, 16)])
            v1l, v1h = _split(tv[b][e, pl.ds(64, 16)])
            v2l, v2h = _split(tv[b][e, pl.ds(80, 16)])
            rb1l, rb1h = _split(rbfv[b][e, pl.ds(0, 16)])
            rb2l, rb2h = _split(rbfv[b][e, pl.ds(16, 16)])
            rb3l, rb3h = _split(rbfv[b][e, pl.ds(32, 16)])
            t1l = xh1l * rb1l
            t1h = xh1h * rb1h
            t2l = xh2l * rb2l
            t2h = xh2h * rb2h
            msg_v[e, pl.ds(0, 16)] = xh3l * rb3l
            msg_v[e, pl.ds(16, 16)] = xh3h * rb3h
            msg_v[e, pl.ds(32, 16)] = t1l * v0l + t2l * ev0
            msg_v[e, pl.ds(48, 16)] = t1h * v0h + t2h * ev0
            msg_v[e, pl.ds(64, 16)] = t1l * v1l + t2l * ev1
            msg_v[e, pl.ds(80, 16)] = t1h * v1h + t2h * ev1
            msg_v[e, pl.ds(96, 16)] = t1l * v2l + t2l * ev2
            msg_v[e, pl.ds(112, 16)] = t1h * v2h + t2h * ev2
            return 0
        lax.fori_loop(0, CHUNK, _edge, 0)
        pltpu.sync_copy(msg_v, acc.at[idxi[b]], add=True)

    def _zfill16(k, _):
        r = k // (MROW // 16)
        col = (k % (MROW // 16)) * 16
        zrow_v[r, pl.ds(col, 16)] = jnp.zeros((16,), jnp.float32)
        return 0
    lax.fori_loop(0, zrows * (MROW // 16), _zfill16, 0)

    for p in range(2):
        g = c * 2 + p

        def _zero(k, _):
            pltpu.sync_copy(zrow_v, acc.at[pl.ds(row0 + k * zrows, zrows), :])
            return 0
        lax.fori_loop(0, n_zcopies, _zero, 0)
        plsc.subcore_barrier()

        issue_idx(0, 0)
        wait_idx(0)
        issue_main(0, 0, g)
        issue_idx(1, 1)

        def _pair(pi, _):
            for bb in range(2):
                ci = pi * 2 + bb
                wait_idx(1 - bb)
                issue_main(ci + 1, 1 - bb, g)
                wait_main(bb)
                compute_scatter(bb)

                @pl.when(ci < n_chunks - 2)
                def _():
                    issue_idx(ci + 2, bb)
            return 0
        lax.fori_loop(0, (n_chunks - 1) // 2, _pair, 0)

        last = (n_chunks - 1) % 2
        if n_chunks % 2 == 0:
            wait_idx(last)
            issue_main(n_chunks - 1, last, g)
            wait_main(1 - last)
            compute_scatter(1 - last)
        wait_main(last)
        compute_scatter(last)

        plsc.subcore_barrier()
        pltpu.sync_copy(acc.at[pl.ds(row0, rows_per_tile), :],
                        out_hbm.at[g, pl.ds(row0, rows_per_tile), :])
        plsc.subcore_barrier()


def _sc_call(t2, rbf_hp, ej, ei, n_pad, n_edges):
    mesh = plsc.VectorSubcoreMesh(core_axis_name="c", subcore_axis_name="s")
    body = functools.partial(_sc_body, n_pad, n_edges)
    dma = pltpu.SemaphoreType.DMA
    return pl.kernel(
        body,
        out_type=jax.ShapeDtypeStruct((G, n_pad, MROW), jnp.float32),
        mesh=mesh,
        compiler_params=pltpu.CompilerParams(use_tc_tiling_on_sc=False),
        scratch_types=[
            pltpu.VMEM_SHARED((n_pad, MROW), jnp.float32),
            pltpu.VMEM((CHUNK,), jnp.int32),
            pltpu.VMEM((CHUNK,), jnp.int32),
            pltpu.VMEM((CHUNK,), jnp.int32),
            pltpu.VMEM((CHUNK,), jnp.int32),
            pltpu.VMEM((CHUNK,), jnp.int32),
            pltpu.VMEM((CHUNK,), jnp.int32),
            pltpu.VMEM((CHUNK, H), jnp.int32),
            pltpu.VMEM((CHUNK, H), jnp.int32),
            pltpu.VMEM((CHUNK, 64), jnp.int32),
            pltpu.VMEM((CHUNK, 64), jnp.int32),
            pltpu.VMEM((CHUNK, MROW), jnp.float32),
            pltpu.VMEM((16, MROW), jnp.float32),
            dma, dma, dma, dma, dma, dma,
        ],
    )(t2, rbf_hp, ej, ei)


def _perm_and_scales():
    p = np.zeros(3 * H, dtype=np.int32)
    s = np.zeros(3 * H, dtype=np.float32)
    inv3 = 1.0 / math.sqrt(3.0)
    invh = 1.0 / math.sqrt(H)
    for g in range(G):
        for j in range(GC):
            half = j // HW
            c = (j % HW) // 16
            l = j % 16
            p[GC * g + j] = c * H + GF * g + 16 * half + l
            s[GC * g + j] = inv3 * invh if c < 2 else inv3
    return p, s


_P, _SFULL = _perm_and_scales()


def kernel(x, vec, edge_index, edge_rbf, edge_vector, W1, b1, W2, b2, We, be):
    n = x.shape[0]
    e = edge_rbf.shape[0]
    p = jnp.asarray(_P)
    sf = jnp.asarray(_SFULL)

    W2P, b2P = W2[p], b2[p]
    WeP = We[p] * sf[:, None]
    beP = be[p] * sf

    x_hp = _node_mlp(x, W1, b1, W2P, b2P)
    ev16 = jnp.pad(edge_vector, ((0, 0), (0, 13)))
    rbf_hp = _edge_proj(edge_rbf, ev16, WeP, beP)

    xb = lax.bitcast_convert_type(x_hp.reshape(n, G, 2, HW), jnp.int32)
    xw = _pack_words(xb[:, :, 0], xb[:, :, 1])
    vb = lax.bitcast_convert_type(vec, jnp.int32)
    vb = vb.reshape(n, 3, G, 2, 16)
    va = vb[:, :, :, 0].transpose(0, 2, 1, 3).reshape(n, G, HW)
    vbb = vb[:, :, :, 1].transpose(0, 2, 1, 3).reshape(n, G, HW)
    vw = _pack_words(va, vbb)
    pad = jnp.zeros((n, G, H - 2 * HW), jnp.int32)
    t2 = jnp.concatenate([xw, vw, pad], axis=2).reshape(n * G, H)

    ej = edge_index[0].astype(jnp.int32)
    ei = edge_index[1].astype(jnp.int32)

    n_pad = ((n + NS * 128 - 1) // (NS * 128)) * (NS * 128)
    outg = _sc_call(t2, rbf_hp, ej, ei, n_pad, e)
    outg = outg[:, :n].reshape(G, n, 4, GF)

    d_x = outg[:, :, 0, :].transpose(1, 0, 2).reshape(n, H)
    d_vec = (outg[:, :, 1:4, :].transpose(1, 2, 0, 3).reshape(n, 3, H))
    return (d_x, d_vec)
